# baseline (speedup 1.0000x reference)
import functools

import jax
import jax.numpy as jnp
from jax import lax
from jax.experimental import pallas as pl
from jax.experimental.pallas import tpu as pltpu
from jax.experimental.pallas import tpu_sc as plsc

Q = 1024
D = 128
NKEYS = 100000
KNN = 16
RADIUS2 = 18.0 * 18.0

BK = 1024
NBLK = 98
KPAD = NBLK * BK
BUCKET = 128
NBUCK = KPAD // BUCKET
NB_LOCAL = BK // BUCKET
NEG_INF = float("-inf")
ISENT = 2**30

NC = 2
NS = 16
NW = NC * NS
GROWS = Q * KNN
ROWS_PER_W = GROWS // NW
CHUNK = 128
NCHUNK = ROWS_PER_W // CHUNK


def _p1_body(q_ref, k_ref, neg_ref, bmax_ref):
    j = pl.program_id(0)
    q = q_ref[...]
    kb = k_ref[...]
    qk = lax.dot_general(q, kb, (((1,), (1,)), ((), ())),
                         preferred_element_type=jnp.float32)
    q2 = jnp.sum(q * q, axis=1, keepdims=True)
    k2 = jnp.sum(kb * kb, axis=1)[None, :]
    neg = 2.0 * qk - (q2 + k2)

    def masked():
        gidx = j * BK + lax.broadcasted_iota(jnp.int32, (Q, BK), 1)
        return jnp.where(gidx < NKEYS, neg, NEG_INF)

    neg = lax.cond(j == NBLK - 1, masked, lambda: neg)
    neg_ref[...] = neg
    maxes = [jnp.max(neg[:, s * BUCKET:(s + 1) * BUCKET], axis=1, keepdims=True)
             for s in range(NB_LOCAL)]
    bmax_ref[...] = jnp.concatenate(maxes, axis=1)[None]


def _phase1(queries, keys):
    return pl.pallas_call(
        _p1_body,
        grid=(NBLK,),
        in_specs=[pl.BlockSpec((Q, D), lambda j: (0, 0)),
                  pl.BlockSpec((BK, D), lambda j: (j, 0))],
        out_specs=[pl.BlockSpec((Q, BK), lambda j: (0, j)),
                   pl.BlockSpec((1, Q, NB_LOCAL), lambda j: (j, 0, 0))],
        out_shape=[jax.ShapeDtypeStruct((Q, KPAD), jnp.float32),
                   jax.ShapeDtypeStruct((NBLK, Q, NB_LOCAL), jnp.float32)],
    )(queries, keys)


def _p2_body(bmax_ref, bid_ref, flat_ref):
    x = bmax_ref[...]
    biota = lax.broadcasted_iota(jnp.int32, (Q, NBUCK), 1)
    cols = []
    for _ in range(KNN):
        m = jnp.max(x, axis=1, keepdims=True)
        b = jnp.min(jnp.where(x == m, biota, ISENT), axis=1, keepdims=True)
        cols.append(b)
        x = jnp.where(biota == b, NEG_INF, x)
    bid = jnp.concatenate(cols, axis=1)
    riota = lax.broadcasted_iota(jnp.int32, (Q, KNN), 0)
    bid_ref[...] = bid
    flat_ref[...] = riota * NBUCK + bid


def _phase2(bmax_flat):
    return pl.pallas_call(
        _p2_body,
        out_shape=[jax.ShapeDtypeStruct((Q, KNN), jnp.int32),
                   jax.ShapeDtypeStruct((Q, KNN), jnp.int32)],
    )(bmax_flat)


@functools.cache
def _sc_gather_fn():
    mesh = plsc.VectorSubcoreMesh(
        core_axis_name="c", subcore_axis_name="s", num_cores=NC)

    @functools.partial(
        pl.kernel,
        mesh=mesh,
        out_type=jax.ShapeDtypeStruct((GROWS, BUCKET), jnp.float32),
        scratch_types=[
            pltpu.VMEM((NCHUNK, CHUNK), jnp.int32),
            pltpu.VMEM((CHUNK, BUCKET), jnp.float32),
            pltpu.SemaphoreType.DMA,
        ],
    )
    def _sc_gather(neg_hbm, idx_hbm, out_hbm, idx_v, rows_v, sem):
        wid = lax.axis_index("s") * NC + lax.axis_index("c")
        pltpu.sync_copy(idx_hbm.at[pl.ds(wid * NCHUNK, NCHUNK)], idx_v)
        for ch in range(NCHUNK):
            pltpu.async_copy(neg_hbm.at[idx_v.at[ch]], rows_v, sem).wait()
            pltpu.sync_copy(
                rows_v,
                out_hbm.at[pl.ds(wid * ROWS_PER_W + ch * CHUNK, CHUNK)])

    return _sc_gather


def _p4_body(g_ref, bid_ref, vals_ref, idx_ref, within_ref):
    x = g_ref[...]
    bid = bid_ref[...]
    lane = lax.broadcasted_iota(jnp.int32, (Q, BUCKET), 1)
    gidx = jnp.concatenate(
        [bid[:, t:t + 1] * BUCKET + lane for t in range(KNN)], axis=1)
    vs, ids = [], []
    for _ in range(KNN):
        m = jnp.max(x, axis=1, keepdims=True)
        ai = jnp.min(jnp.where(x == m, gidx, ISENT), axis=1, keepdims=True)
        vs.append(m)
        ids.append(ai)
        x = jnp.where(gidx == ai, NEG_INF, x)
    vals = jnp.concatenate(vs, axis=1)
    idx = jnp.concatenate(ids, axis=1)
    within = vals >= -RADIUS2
    vals_ref[...] = jnp.where(within, vals, -RADIUS2)
    idx_ref[...] = idx
    within_ref[...] = within.astype(jnp.int32)


def _phase4(gathered, bid):
    return pl.pallas_call(
        _p4_body,
        out_shape=[jax.ShapeDtypeStruct((Q, KNN), jnp.float32),
                   jax.ShapeDtypeStruct((Q, KNN), jnp.int32),
                   jax.ShapeDtypeStruct((Q, KNN), jnp.int32)],
    )(gathered, bid)


_ABLATE = 0


def kernel(queries, keys):
    neg, bmax = _phase1(queries, keys)
    if _ABLATE == 1:
        return neg, bmax
    bmax_flat = bmax.transpose(1, 0, 2).reshape(Q, NBUCK)
    bid, flat = _phase2(bmax_flat)
    gathered = _sc_gather_fn()(neg.reshape(Q * NBUCK, BUCKET),
                               flat.reshape(NW * NCHUNK, CHUNK))
    if _ABLATE == 3:
        return gathered, bid
    vals, idx, within = _phase4(gathered.reshape(Q, KNN * BUCKET), bid)
    return vals, idx, within


# speedup vs baseline: 1.1438x; 1.1438x over previous
import functools

import jax
import jax.numpy as jnp
from jax import lax
from jax.experimental import pallas as pl
from jax.experimental.pallas import tpu as pltpu
from jax.experimental.pallas import tpu_sc as plsc

Q = 1024
D = 128
NKEYS = 100000
KNN = 16
RADIUS2 = 18.0 * 18.0

BK = 1024
NBLK = 98
KPAD = NBLK * BK
BUCKET = 128
NBUCK = KPAD // BUCKET
NB_LOCAL = BK // BUCKET
NEG_INF = float("-inf")
ISENT = 2**30

NC = 2
NS = 16
NW = NC * NS
GROWS = Q * KNN
ROWS_PER_W = GROWS // NW
CHUNK = 128
NCHUNK = ROWS_PER_W // CHUNK


def _p1_body(q_ref, k_ref, neg_ref, bmax_ref):
    j = pl.program_id(0)
    q = q_ref[...]
    kb = k_ref[...]
    qk = lax.dot_general(q, kb, (((1,), (1,)), ((), ())),
                         preferred_element_type=jnp.float32)
    q2 = jnp.sum(q * q, axis=1, keepdims=True)
    k2 = jnp.sum(kb * kb, axis=1)[None, :]
    neg = -((q2 + k2) - 2.0 * qk)
    valid = (j * BK + lax.broadcasted_iota(jnp.int32, (1, BK), 1)) < NKEYS
    neg = jnp.where(valid, neg, NEG_INF)
    neg_ref[...] = neg
    bmax_ref[...] = jnp.max(neg.reshape(Q, NB_LOCAL, BUCKET), axis=2)[None]


def _phase1(queries, keys):
    return pl.pallas_call(
        _p1_body,
        grid=(NBLK,),
        in_specs=[pl.BlockSpec((Q, D), lambda j: (0, 0)),
                  pl.BlockSpec((BK, D), lambda j: (j, 0))],
        out_specs=[pl.BlockSpec((Q, BK), lambda j: (0, j)),
                   pl.BlockSpec((1, Q, NB_LOCAL), lambda j: (j, 0, 0))],
        out_shape=[jax.ShapeDtypeStruct((Q, KPAD), jnp.float32),
                   jax.ShapeDtypeStruct((NBLK, Q, NB_LOCAL), jnp.float32)],
    )(queries, keys)


def _p2_body(bmax_ref, bid_ref, flat_ref):
    x = bmax_ref[...]
    biota = lax.broadcasted_iota(jnp.int32, (Q, NBUCK), 1)
    cols = []
    for _ in range(KNN):
        m = jnp.max(x, axis=1, keepdims=True)
        b = jnp.min(jnp.where(x == m, biota, ISENT), axis=1, keepdims=True)
        cols.append(b)
        x = jnp.where(biota == b, NEG_INF, x)
    bid = jnp.concatenate(cols, axis=1)
    riota = lax.broadcasted_iota(jnp.int32, (Q, KNN), 0)
    bid_ref[...] = bid
    flat_ref[...] = riota * NBUCK + bid


def _phase2(bmax_flat):
    return pl.pallas_call(
        _p2_body,
        out_shape=[jax.ShapeDtypeStruct((Q, KNN), jnp.int32),
                   jax.ShapeDtypeStruct((Q, KNN), jnp.int32)],
    )(bmax_flat)


@functools.cache
def _sc_gather_fn():
    mesh = plsc.VectorSubcoreMesh(
        core_axis_name="c", subcore_axis_name="s", num_cores=NC)

    @functools.partial(
        pl.kernel,
        mesh=mesh,
        out_type=jax.ShapeDtypeStruct((GROWS, BUCKET), jnp.float32),
        scratch_types=[
            pltpu.VMEM((NCHUNK, CHUNK), jnp.int32),
            pltpu.VMEM((CHUNK, BUCKET), jnp.float32),
            pltpu.SemaphoreType.DMA,
        ],
    )
    def _sc_gather(neg_hbm, idx_hbm, out_hbm, idx_v, rows_v, sem):
        wid = lax.axis_index("s") * NC + lax.axis_index("c")
        pltpu.sync_copy(idx_hbm.at[pl.ds(wid * NCHUNK, NCHUNK)], idx_v)
        for ch in range(NCHUNK):
            pltpu.async_copy(neg_hbm.at[idx_v.at[ch]], rows_v, sem).wait()
            pltpu.sync_copy(
                rows_v,
                out_hbm.at[pl.ds(wid * ROWS_PER_W + ch * CHUNK, CHUNK)])

    return _sc_gather


def _p4_body(g_ref, bid_ref, vals_ref, idx_ref, within_ref):
    x = g_ref[...]
    bid = bid_ref[...]
    lane = lax.broadcasted_iota(jnp.int32, (Q, BUCKET), 1)
    gidx = jnp.concatenate(
        [bid[:, t:t + 1] * BUCKET + lane for t in range(KNN)], axis=1)
    vs, ids = [], []
    for _ in range(KNN):
        m = jnp.max(x, axis=1, keepdims=True)
        ai = jnp.min(jnp.where(x == m, gidx, ISENT), axis=1, keepdims=True)
        vs.append(m)
        ids.append(ai)
        x = jnp.where(gidx == ai, NEG_INF, x)
    vals = jnp.concatenate(vs, axis=1)
    idx = jnp.concatenate(ids, axis=1)
    within = vals >= -RADIUS2
    vals_ref[...] = jnp.where(within, vals, -RADIUS2)
    idx_ref[...] = idx
    within_ref[...] = within.astype(jnp.int32)


def _phase4(gathered, bid):
    return pl.pallas_call(
        _p4_body,
        out_shape=[jax.ShapeDtypeStruct((Q, KNN), jnp.float32),
                   jax.ShapeDtypeStruct((Q, KNN), jnp.int32),
                   jax.ShapeDtypeStruct((Q, KNN), jnp.int32)],
    )(gathered, bid)


_ABLATE = 0


def kernel(queries, keys):
    neg, bmax = _phase1(queries, keys)
    if _ABLATE == 1:
        return neg, bmax
    bmax_flat = bmax.transpose(1, 0, 2).reshape(Q, NBUCK)
    bid, flat = _phase2(bmax_flat)
    gathered = _sc_gather_fn()(neg.reshape(Q * NBUCK, BUCKET),
                               flat.reshape(NW * NCHUNK, CHUNK))
    if _ABLATE == 3:
        return gathered, bid
    vals, idx, within = _phase4(gathered.reshape(Q, KNN * BUCKET), bid)
    return vals, idx, within


# speedup vs baseline: 1.6852x; 1.4734x over previous
import functools

import jax
import jax.numpy as jnp
from jax import lax
from jax.experimental import pallas as pl
from jax.experimental.pallas import tpu as pltpu
from jax.experimental.pallas import tpu_sc as plsc

Q = 1024
D = 128
NKEYS = 100000
KNN = 16
RADIUS2 = 18.0 * 18.0

BK = 1024
NBLK = 98
KPAD = NBLK * BK
BUCKET = 128
NBUCK = KPAD // BUCKET
NB_LOCAL = BK // BUCKET
NEG_INF = float("-inf")
ISENT = 2**30

NC = 2
NS = 16
NW = NC * NS
GROWS = Q * KNN
ROWS_PER_W = GROWS // NW
CHUNK = 128
NCHUNK = ROWS_PER_W // CHUNK
BROWS = Q * NB_LOCAL


def _p1_body(q_ref, k_ref, neg_ref, bmax_ref):
    j = pl.program_id(0)
    q = q_ref[...]
    kb = k_ref[...]
    qk = lax.dot_general(q, kb, (((1,), (1,)), ((), ())),
                         preferred_element_type=jnp.float32)
    q2 = jnp.sum(q * q, axis=1, keepdims=True)
    k2 = jnp.sum(kb * kb, axis=1)[None, :]
    neg = -((q2 + k2) - 2.0 * qk)
    valid = (j * BK + lax.broadcasted_iota(jnp.int32, (1, BK), 1)) < NKEYS
    neg = jnp.where(valid, neg, NEG_INF)
    neg_ref[...] = neg.reshape(BROWS, BUCKET)
    bmax_ref[...] = jnp.max(neg.reshape(Q, NB_LOCAL, BUCKET), axis=2)[None]


def _phase1(queries, keys):
    return pl.pallas_call(
        _p1_body,
        grid=(NBLK,),
        in_specs=[pl.BlockSpec((Q, D), lambda j: (0, 0)),
                  pl.BlockSpec((BK, D), lambda j: (j, 0))],
        out_specs=[pl.BlockSpec((BROWS, BUCKET), lambda j: (j, 0)),
                   pl.BlockSpec((1, Q, NB_LOCAL), lambda j: (j, 0, 0))],
        out_shape=[jax.ShapeDtypeStruct((NBLK * BROWS, BUCKET), jnp.float32),
                   jax.ShapeDtypeStruct((NBLK, Q, NB_LOCAL), jnp.float32)],
    )(queries, keys)


def _p2_body(bmax_ref, bid_ref, flat_ref):
    x = bmax_ref[...]
    biota = lax.broadcasted_iota(jnp.int32, (Q, NBUCK), 1)
    cols = []
    for _ in range(KNN):
        m = jnp.max(x, axis=1, keepdims=True)
        b = jnp.min(jnp.where(x == m, biota, ISENT), axis=1, keepdims=True)
        cols.append(b)
        x = jnp.where(biota == b, NEG_INF, x)
    bid = jnp.concatenate(cols, axis=1)
    riota = lax.broadcasted_iota(jnp.int32, (Q, KNN), 0)
    bid_ref[...] = bid
    j8 = bid // NB_LOCAL
    flat_ref[...] = j8 * BROWS + riota * NB_LOCAL + (bid - j8 * NB_LOCAL)


def _phase2(bmax_flat):
    return pl.pallas_call(
        _p2_body,
        out_shape=[jax.ShapeDtypeStruct((Q, KNN), jnp.int32),
                   jax.ShapeDtypeStruct((Q, KNN), jnp.int32)],
    )(bmax_flat)


@functools.cache
def _sc_gather_fn():
    mesh = plsc.VectorSubcoreMesh(
        core_axis_name="c", subcore_axis_name="s", num_cores=NC)

    @functools.partial(
        pl.kernel,
        mesh=mesh,
        out_type=jax.ShapeDtypeStruct((GROWS, BUCKET), jnp.float32),
        scratch_types=[
            pltpu.VMEM((NCHUNK, CHUNK), jnp.int32),
            pltpu.VMEM((CHUNK, BUCKET), jnp.float32),
            pltpu.SemaphoreType.DMA,
        ],
    )
    def _sc_gather(neg_hbm, idx_hbm, out_hbm, idx_v, rows_v, sem):
        wid = lax.axis_index("s") * NC + lax.axis_index("c")
        pltpu.sync_copy(idx_hbm.at[pl.ds(wid * NCHUNK, NCHUNK)], idx_v)
        for ch in range(NCHUNK):
            pltpu.async_copy(neg_hbm.at[idx_v.at[ch]], rows_v, sem).wait()
            pltpu.sync_copy(
                rows_v,
                out_hbm.at[pl.ds(wid * ROWS_PER_W + ch * CHUNK, CHUNK)])

    return _sc_gather


def _p4_body(g_ref, bid_ref, vals_ref, idx_ref, within_ref):
    x = g_ref[...]
    bid = bid_ref[...]
    lane = lax.broadcasted_iota(jnp.int32, (Q, BUCKET), 1)
    gidx = jnp.concatenate(
        [bid[:, t:t + 1] * BUCKET + lane for t in range(KNN)], axis=1)
    vs, ids = [], []
    for _ in range(KNN):
        m = jnp.max(x, axis=1, keepdims=True)
        ai = jnp.min(jnp.where(x == m, gidx, ISENT), axis=1, keepdims=True)
        vs.append(m)
        ids.append(ai)
        x = jnp.where(gidx == ai, NEG_INF, x)
    vals = jnp.concatenate(vs, axis=1)
    idx = jnp.concatenate(ids, axis=1)
    within = vals >= -RADIUS2
    vals_ref[...] = jnp.where(within, vals, -RADIUS2)
    idx_ref[...] = idx
    within_ref[...] = within.astype(jnp.int32)


def _phase4(gathered, bid):
    return pl.pallas_call(
        _p4_body,
        out_shape=[jax.ShapeDtypeStruct((Q, KNN), jnp.float32),
                   jax.ShapeDtypeStruct((Q, KNN), jnp.int32),
                   jax.ShapeDtypeStruct((Q, KNN), jnp.int32)],
    )(gathered, bid)


_ABLATE = 0


def kernel(queries, keys):
    neg, bmax = _phase1(queries, keys)
    if _ABLATE == 1:
        return neg, bmax
    bmax_flat = bmax.transpose(1, 0, 2).reshape(Q, NBUCK)
    bid, flat = _phase2(bmax_flat)
    gathered = _sc_gather_fn()(neg, flat.reshape(NW * NCHUNK, CHUNK))
    if _ABLATE == 3:
        return gathered, bid
    vals, idx, within = _phase4(gathered.reshape(Q, KNN * BUCKET), bid)
    return vals, idx, within


# speedup vs baseline: 1.7744x; 1.0529x over previous
import functools

import jax
import jax.numpy as jnp
from jax import lax
from jax.experimental import pallas as pl
from jax.experimental.pallas import tpu as pltpu
from jax.experimental.pallas import tpu_sc as plsc

Q = 1024
D = 128
NKEYS = 100000
KNN = 16
RADIUS2 = 18.0 * 18.0

BK = 2048
NBLK = 49
KPAD = NBLK * BK
BUCKET = 128
NBUCK = KPAD // BUCKET
NB_LOCAL = BK // BUCKET
NEG_INF = float("-inf")
ISENT = 2**30

NC = 2
NS = 16
NW = NC * NS
GROWS = Q * KNN
ROWS_PER_W = GROWS // NW
CHUNK = 128
NCHUNK = ROWS_PER_W // CHUNK
BROWS = Q * NB_LOCAL


def _p1_body(q_ref, k_ref, neg_ref, bmax_ref):
    j = pl.program_id(0)
    q = q_ref[...]
    kb = k_ref[...]
    qk = lax.dot_general(q, kb, (((1,), (1,)), ((), ())),
                         preferred_element_type=jnp.float32)
    q2 = jnp.sum(q * q, axis=1, keepdims=True)
    k2 = jnp.sum(kb * kb, axis=1)[None, :]
    neg = -((q2 + k2) - 2.0 * qk)
    valid = (j * BK + lax.broadcasted_iota(jnp.int32, (1, BK), 1)) < NKEYS
    neg = jnp.where(valid, neg, NEG_INF)
    neg_ref[...] = neg.reshape(BROWS, BUCKET)
    bmax_ref[...] = jnp.max(neg.reshape(Q, NB_LOCAL, BUCKET), axis=2)[None]


def _phase1(queries, keys):
    return pl.pallas_call(
        _p1_body,
        grid=(NBLK,),
        in_specs=[pl.BlockSpec((Q, D), lambda j: (0, 0)),
                  pl.BlockSpec((BK, D), lambda j: (j, 0))],
        out_specs=[pl.BlockSpec((BROWS, BUCKET), lambda j: (j, 0)),
                   pl.BlockSpec((1, Q, NB_LOCAL), lambda j: (j, 0, 0))],
        out_shape=[jax.ShapeDtypeStruct((NBLK * BROWS, BUCKET), jnp.float32),
                   jax.ShapeDtypeStruct((NBLK, Q, NB_LOCAL), jnp.float32)],
    )(queries, keys)


def _p2_body(bmax_ref, bid_ref, flat_ref):
    x = bmax_ref[...]
    biota = lax.broadcasted_iota(jnp.int32, (Q, NBUCK), 1)
    cols = []
    for _ in range(KNN):
        m = jnp.max(x, axis=1, keepdims=True)
        b = jnp.min(jnp.where(x == m, biota, ISENT), axis=1, keepdims=True)
        cols.append(b)
        x = jnp.where(biota == b, NEG_INF, x)
    bid = jnp.concatenate(cols, axis=1)
    riota = lax.broadcasted_iota(jnp.int32, (Q, KNN), 0)
    bid_ref[...] = bid
    j8 = bid // NB_LOCAL
    flat_ref[...] = j8 * BROWS + riota * NB_LOCAL + (bid - j8 * NB_LOCAL)


def _phase2(bmax_flat):
    return pl.pallas_call(
        _p2_body,
        out_shape=[jax.ShapeDtypeStruct((Q, KNN), jnp.int32),
                   jax.ShapeDtypeStruct((Q, KNN), jnp.int32)],
    )(bmax_flat)


@functools.cache
def _sc_gather_fn():
    mesh = plsc.VectorSubcoreMesh(
        core_axis_name="c", subcore_axis_name="s", num_cores=NC)

    @functools.partial(
        pl.kernel,
        mesh=mesh,
        out_type=jax.ShapeDtypeStruct((GROWS, BUCKET), jnp.float32),
        scratch_types=[
            pltpu.VMEM((NCHUNK, CHUNK), jnp.int32),
            pltpu.VMEM((CHUNK, BUCKET), jnp.float32),
            pltpu.SemaphoreType.DMA,
        ],
    )
    def _sc_gather(neg_hbm, idx_hbm, out_hbm, idx_v, rows_v, sem):
        wid = lax.axis_index("s") * NC + lax.axis_index("c")
        pltpu.sync_copy(idx_hbm.at[pl.ds(wid * NCHUNK, NCHUNK)], idx_v)
        for ch in range(NCHUNK):
            pltpu.async_copy(neg_hbm.at[idx_v.at[ch]], rows_v, sem).wait()
            pltpu.sync_copy(
                rows_v,
                out_hbm.at[pl.ds(wid * ROWS_PER_W + ch * CHUNK, CHUNK)])

    return _sc_gather


def _p4_body(g_ref, bid_ref, vals_ref, idx_ref, within_ref):
    x = g_ref[...]
    bid = bid_ref[...]
    lane = lax.broadcasted_iota(jnp.int32, (Q, BUCKET), 1)
    gidx = jnp.concatenate(
        [bid[:, t:t + 1] * BUCKET + lane for t in range(KNN)], axis=1)
    vs, ids = [], []
    for _ in range(KNN):
        m = jnp.max(x, axis=1, keepdims=True)
        ai = jnp.min(jnp.where(x == m, gidx, ISENT), axis=1, keepdims=True)
        vs.append(m)
        ids.append(ai)
        x = jnp.where(gidx == ai, NEG_INF, x)
    vals = jnp.concatenate(vs, axis=1)
    idx = jnp.concatenate(ids, axis=1)
    within = vals >= -RADIUS2
    vals_ref[...] = jnp.where(within, vals, -RADIUS2)
    idx_ref[...] = idx
    within_ref[...] = within.astype(jnp.int32)


def _phase4(gathered, bid):
    return pl.pallas_call(
        _p4_body,
        out_shape=[jax.ShapeDtypeStruct((Q, KNN), jnp.float32),
                   jax.ShapeDtypeStruct((Q, KNN), jnp.int32),
                   jax.ShapeDtypeStruct((Q, KNN), jnp.int32)],
    )(gathered, bid)


_ABLATE = 0


def kernel(queries, keys):
    neg, bmax = _phase1(queries, keys)
    if _ABLATE == 1:
        return neg, bmax
    bmax_flat = bmax.transpose(1, 0, 2).reshape(Q, NBUCK)
    bid, flat = _phase2(bmax_flat)
    gathered = _sc_gather_fn()(neg, flat.reshape(NW * NCHUNK, CHUNK))
    if _ABLATE == 3:
        return gathered, bid
    vals, idx, within = _phase4(gathered.reshape(Q, KNN * BUCKET), bid)
    return vals, idx, within


# speedup vs baseline: 1.7772x; 1.0016x over previous
import functools

import jax
import jax.numpy as jnp
from jax import lax
from jax.experimental import pallas as pl
from jax.experimental.pallas import tpu as pltpu
from jax.experimental.pallas import tpu_sc as plsc

Q = 1024
D = 128
NKEYS = 100000
KNN = 16
RADIUS2 = 18.0 * 18.0

BK = 4096
NBLK = 25
KPAD = NBLK * BK
BUCKET = 128
NBUCK = KPAD // BUCKET
NB_LOCAL = BK // BUCKET
NEG_INF = float("-inf")
ISENT = 2**30

NC = 2
NS = 16
NW = NC * NS
GROWS = Q * KNN
ROWS_PER_W = GROWS // NW
CHUNK = 128
NCHUNK = ROWS_PER_W // CHUNK
BROWS = Q * NB_LOCAL


def _p1_body(q_ref, k_ref, neg_ref, bmax_ref):
    j = pl.program_id(0)
    q = q_ref[...]
    kb = k_ref[...]
    qk = lax.dot_general(q, kb, (((1,), (1,)), ((), ())),
                         preferred_element_type=jnp.float32)
    q2 = jnp.sum(q * q, axis=1, keepdims=True)
    k2 = jnp.sum(kb * kb, axis=1)[None, :]
    neg = -((q2 + k2) - 2.0 * qk)
    valid = (j * BK + lax.broadcasted_iota(jnp.int32, (1, BK), 1)) < NKEYS
    neg = jnp.where(valid, neg, NEG_INF)
    neg_ref[...] = neg.reshape(BROWS, BUCKET)
    bmax_ref[...] = jnp.max(neg.reshape(Q, NB_LOCAL, BUCKET), axis=2)[None]


def _phase1(queries, keys):
    return pl.pallas_call(
        _p1_body,
        grid=(NBLK,),
        in_specs=[pl.BlockSpec((Q, D), lambda j: (0, 0)),
                  pl.BlockSpec((BK, D), lambda j: (j, 0))],
        out_specs=[pl.BlockSpec((BROWS, BUCKET), lambda j: (j, 0)),
                   pl.BlockSpec((1, Q, NB_LOCAL), lambda j: (j, 0, 0))],
        out_shape=[jax.ShapeDtypeStruct((NBLK * BROWS, BUCKET), jnp.float32),
                   jax.ShapeDtypeStruct((NBLK, Q, NB_LOCAL), jnp.float32)],
    )(queries, keys)


def _p2_body(bmax_ref, bid_ref, flat_ref):
    x = bmax_ref[...]
    biota = lax.broadcasted_iota(jnp.int32, (Q, NBUCK), 1)
    cols = []
    for _ in range(KNN):
        m = jnp.max(x, axis=1, keepdims=True)
        b = jnp.min(jnp.where(x == m, biota, ISENT), axis=1, keepdims=True)
        cols.append(b)
        x = jnp.where(biota == b, NEG_INF, x)
    bid = jnp.concatenate(cols, axis=1)
    riota = lax.broadcasted_iota(jnp.int32, (Q, KNN), 0)
    bid_ref[...] = bid
    j8 = bid // NB_LOCAL
    flat_ref[...] = j8 * BROWS + riota * NB_LOCAL + (bid - j8 * NB_LOCAL)


def _phase2(bmax_flat):
    return pl.pallas_call(
        _p2_body,
        out_shape=[jax.ShapeDtypeStruct((Q, KNN), jnp.int32),
                   jax.ShapeDtypeStruct((Q, KNN), jnp.int32)],
    )(bmax_flat)


@functools.cache
def _sc_gather_fn():
    mesh = plsc.VectorSubcoreMesh(
        core_axis_name="c", subcore_axis_name="s", num_cores=NC)

    @functools.partial(
        pl.kernel,
        mesh=mesh,
        out_type=jax.ShapeDtypeStruct((GROWS, BUCKET), jnp.float32),
        scratch_types=[
            pltpu.VMEM((NCHUNK, CHUNK), jnp.int32),
            pltpu.VMEM((CHUNK, BUCKET), jnp.float32),
            pltpu.SemaphoreType.DMA,
        ],
    )
    def _sc_gather(neg_hbm, idx_hbm, out_hbm, idx_v, rows_v, sem):
        wid = lax.axis_index("s") * NC + lax.axis_index("c")
        pltpu.sync_copy(idx_hbm.at[pl.ds(wid * NCHUNK, NCHUNK)], idx_v)
        for ch in range(NCHUNK):
            pltpu.async_copy(neg_hbm.at[idx_v.at[ch]], rows_v, sem).wait()
            pltpu.sync_copy(
                rows_v,
                out_hbm.at[pl.ds(wid * ROWS_PER_W + ch * CHUNK, CHUNK)])

    return _sc_gather


def _p4_body(g_ref, bid_ref, vals_ref, idx_ref, within_ref):
    x = g_ref[...]
    bid = bid_ref[...]
    lane = lax.broadcasted_iota(jnp.int32, (Q, BUCKET), 1)
    gidx = jnp.concatenate(
        [bid[:, t:t + 1] * BUCKET + lane for t in range(KNN)], axis=1)
    vs, ids = [], []
    for _ in range(KNN):
        m = jnp.max(x, axis=1, keepdims=True)
        ai = jnp.min(jnp.where(x == m, gidx, ISENT), axis=1, keepdims=True)
        vs.append(m)
        ids.append(ai)
        x = jnp.where(gidx == ai, NEG_INF, x)
    vals = jnp.concatenate(vs, axis=1)
    idx = jnp.concatenate(ids, axis=1)
    within = vals >= -RADIUS2
    vals_ref[...] = jnp.where(within, vals, -RADIUS2)
    idx_ref[...] = idx
    within_ref[...] = within.astype(jnp.int32)


def _phase4(gathered, bid):
    return pl.pallas_call(
        _p4_body,
        out_shape=[jax.ShapeDtypeStruct((Q, KNN), jnp.float32),
                   jax.ShapeDtypeStruct((Q, KNN), jnp.int32),
                   jax.ShapeDtypeStruct((Q, KNN), jnp.int32)],
    )(gathered, bid)


_ABLATE = 0


def kernel(queries, keys):
    neg, bmax = _phase1(queries, keys)
    if _ABLATE == 1:
        return neg, bmax
    bmax_flat = bmax.transpose(1, 0, 2).reshape(Q, NBUCK)
    bid, flat = _phase2(bmax_flat)
    gathered = _sc_gather_fn()(neg, flat.reshape(NW * NCHUNK, CHUNK))
    if _ABLATE == 3:
        return gathered, bid
    vals, idx, within = _phase4(gathered.reshape(Q, KNN * BUCKET), bid)
    return vals, idx, within


# speedup vs baseline: 1.8333x; 1.0316x over previous
import functools

import jax
import jax.numpy as jnp
from jax import lax
from jax.experimental import pallas as pl
from jax.experimental.pallas import tpu as pltpu
from jax.experimental.pallas import tpu_sc as plsc

Q = 1024
D = 128
NKEYS = 100000
KNN = 16
RADIUS2 = 18.0 * 18.0

BK = 2048
NBLK = 49
KPAD = NBLK * BK
BUCKET = 128
NBUCK = KPAD // BUCKET
NB_LOCAL = BK // BUCKET
NEG_INF = float("-inf")
ISENT = 2**30

NC = 2
NS = 16
NW = NC * NS
GROWS = Q * KNN
ROWS_PER_W = GROWS // NW
CHUNK = 128
NCHUNK = ROWS_PER_W // CHUNK
BROWS = Q * NB_LOCAL


def _p1_body(q_ref, k_ref, neg_ref, bmax_ref):
    j = pl.program_id(0)
    q = q_ref[...]
    kb = k_ref[...]
    qk = lax.dot_general(q, kb, (((1,), (1,)), ((), ())),
                         preferred_element_type=jnp.float32)
    q2 = jnp.sum(q * q, axis=1, keepdims=True)
    k2 = jnp.sum(kb * kb, axis=1)[None, :]
    neg = -((q2 + k2) - 2.0 * qk)
    valid = (j * BK + lax.broadcasted_iota(jnp.int32, (1, BK), 1)) < NKEYS
    neg = jnp.where(valid, neg, NEG_INF)
    neg_ref[...] = neg.reshape(BROWS, BUCKET)
    bmax_ref[...] = jnp.max(neg.reshape(Q, NB_LOCAL, BUCKET), axis=2).T


def _phase1(queries, keys):
    return pl.pallas_call(
        _p1_body,
        grid=(NBLK,),
        in_specs=[pl.BlockSpec((Q, D), lambda j: (0, 0)),
                  pl.BlockSpec((BK, D), lambda j: (j, 0))],
        out_specs=[pl.BlockSpec((BROWS, BUCKET), lambda j: (j, 0)),
                   pl.BlockSpec((NB_LOCAL, Q), lambda j: (j, 0))],
        out_shape=[jax.ShapeDtypeStruct((NBLK * BROWS, BUCKET), jnp.float32),
                   jax.ShapeDtypeStruct((NBUCK, Q), jnp.float32)],
    )(queries, keys)


def _p2_body(bmax_ref, bid_ref, flat_ref):
    x = bmax_ref[...]
    biota = lax.broadcasted_iota(jnp.int32, (NBUCK, Q), 0)
    rows = []
    for _ in range(KNN):
        m = jnp.max(x, axis=0, keepdims=True)
        b = jnp.min(jnp.where(x == m, biota, ISENT), axis=0, keepdims=True)
        rows.append(b)
        x = jnp.where(biota == b, NEG_INF, x)
    bid_t = jnp.concatenate(rows, axis=0)
    qiota = lax.broadcasted_iota(jnp.int32, (KNN, Q), 1)
    bid_ref[...] = bid_t
    j8 = bid_t // NB_LOCAL
    flat_ref[...] = j8 * BROWS + qiota * NB_LOCAL + (bid_t - j8 * NB_LOCAL)


def _phase2(bmax_flat):
    return pl.pallas_call(
        _p2_body,
        out_shape=[jax.ShapeDtypeStruct((KNN, Q), jnp.int32),
                   jax.ShapeDtypeStruct((KNN, Q), jnp.int32)],
    )(bmax_flat)


@functools.cache
def _sc_gather_fn():
    mesh = plsc.VectorSubcoreMesh(
        core_axis_name="c", subcore_axis_name="s", num_cores=NC)

    @functools.partial(
        pl.kernel,
        mesh=mesh,
        out_type=jax.ShapeDtypeStruct((GROWS, BUCKET), jnp.float32),
        scratch_types=[
            pltpu.VMEM((NCHUNK, CHUNK), jnp.int32),
            pltpu.VMEM((NCHUNK, CHUNK, BUCKET), jnp.float32),
            pltpu.SemaphoreType.DMA,
        ],
    )
    def _sc_gather(neg_hbm, idx_hbm, out_hbm, idx_v, rows_v, sem):
        wid = lax.axis_index("s") * NC + lax.axis_index("c")
        pltpu.sync_copy(idx_hbm.at[pl.ds(wid * NCHUNK, NCHUNK)], idx_v)
        copies = [pltpu.async_copy(neg_hbm.at[idx_v.at[ch]], rows_v.at[ch], sem)
                  for ch in range(NCHUNK)]
        for ch in range(NCHUNK):
            copies[ch].wait()
            pltpu.sync_copy(
                rows_v.at[ch],
                out_hbm.at[pl.ds(wid * ROWS_PER_W + ch * CHUNK, CHUNK)])

    return _sc_gather


def _p4_body(g_ref, bid_ref, vals_ref, idx_ref, within_ref):
    x = g_ref[...]
    bid = bid_ref[...]
    lane = lax.broadcasted_iota(jnp.int32, (Q, BUCKET), 1)
    gidx = jnp.concatenate(
        [bid[:, t:t + 1] * BUCKET + lane for t in range(KNN)], axis=1)
    vs, ids = [], []
    for _ in range(KNN):
        m = jnp.max(x, axis=1, keepdims=True)
        ai = jnp.min(jnp.where(x == m, gidx, ISENT), axis=1, keepdims=True)
        vs.append(m)
        ids.append(ai)
        x = jnp.where(gidx == ai, NEG_INF, x)
    vals = jnp.concatenate(vs, axis=1)
    idx = jnp.concatenate(ids, axis=1)
    within = vals >= -RADIUS2
    vals_ref[...] = jnp.where(within, vals, -RADIUS2)
    idx_ref[...] = idx
    within_ref[...] = within.astype(jnp.int32)


def _phase4(gathered, bid):
    return pl.pallas_call(
        _p4_body,
        out_shape=[jax.ShapeDtypeStruct((Q, KNN), jnp.float32),
                   jax.ShapeDtypeStruct((Q, KNN), jnp.int32),
                   jax.ShapeDtypeStruct((Q, KNN), jnp.int32)],
    )(gathered, bid)


_ABLATE = 0


def kernel(queries, keys):
    neg, bmax = _phase1(queries, keys)
    if _ABLATE == 1:
        return neg, bmax
    bid_t, flat_t = _phase2(bmax)
    bid = bid_t.T
    flat = flat_t.T
    gathered = _sc_gather_fn()(neg, flat.reshape(NW * NCHUNK, CHUNK))
    if _ABLATE == 3:
        return gathered, bid
    vals, idx, within = _phase4(gathered.reshape(Q, KNN * BUCKET), bid)
    return vals, idx, within
